# baseline (speedup 1.0000x reference)
import jax
import jax.numpy as jnp
from jax import lax
from jax.experimental import pallas as pl
from jax.experimental.pallas import tpu as pltpu
from jax.experimental.pallas import tpu_sc as plsc

NUM_CORES = 2
NUM_SUBCORES = 16
NW = NUM_CORES * NUM_SUBCORES

B_ROWS = 16384
SEQ = 50
FEATURES = 32
NUM_EMB = 1000000
LANES = 16

TILE = 128
FULL_TILES = NUM_EMB // TILE
TAIL = NUM_EMB - FULL_TILES * TILE
TILES_PER_W = -(-FULL_TILES // NW)

ROWS_PER_W = B_ROWS // NW
CH_ROWS = 16
CHUNK = CH_ROWS * SEQ
NCHUNK = ROWS_PER_W // CH_ROWS
NBUF = 2


def _compact_body(embT_hbm, tail_hbm, out_hbm, src_v, lin_v, in_sem, out_sem):
    wid = lax.axis_index("s") * NUM_CORES + lax.axis_index("c")
    t0 = wid * TILES_PER_W
    t1 = jnp.minimum(t0 + TILES_PER_W, FULL_TILES)
    iota = lax.iota(jnp.int32, LANES)

    def transpose_cols(nrows, lb):
        def col(ic, carry):
            for h in range(2):
                vals = plsc.load_gather(
                    src_v,
                    [iota + h * LANES, jnp.full((LANES,), ic, jnp.int32)])
                lin_v[lb, pl.ds(ic * FEATURES + h * LANES, LANES)] = vals
            return carry
        lax.fori_loop(0, nrows, col, 0)

    def body(i, carry):
        t = t0 + i

        @pl.when(t < t1)
        def _():
            lb = i % 2
            pltpu.make_async_copy(
                embT_hbm.at[:, pl.ds(t * TILE, TILE)], src_v,
                in_sem).start()
            pltpu.make_async_copy(
                embT_hbm.at[:, pl.ds(0, TILE)], src_v, in_sem).wait()
            transpose_cols(TILE, lb)

            @pl.when(i >= 2)
            def _():
                pltpu.make_async_copy(
                    lin_v.at[lb], out_hbm.at[pl.ds(0, TILE * FEATURES)],
                    out_sem).wait()

            pltpu.make_async_copy(
                lin_v.at[lb],
                out_hbm.at[pl.ds(t * TILE * FEATURES, TILE * FEATURES)],
                out_sem).start()
        return carry

    lax.fori_loop(0, TILES_PER_W, body, 0)

    n_tiles = t1 - t0

    @pl.when(n_tiles >= 2)
    def _():
        pltpu.make_async_copy(
            lin_v.at[0], out_hbm.at[pl.ds(0, TILE * FEATURES)],
            out_sem).wait()

    @pl.when(n_tiles >= 1)
    def _():
        pltpu.make_async_copy(
            lin_v.at[0], out_hbm.at[pl.ds(0, TILE * FEATURES)],
            out_sem).wait()

    @pl.when(wid == 0)
    def _():
        pltpu.make_async_copy(
            tail_hbm, lin_v.at[0, pl.ds(0, TAIL * FEATURES)],
            in_sem).start()
        pltpu.make_async_copy(
            tail_hbm, lin_v.at[0, pl.ds(0, TAIL * FEATURES)],
            in_sem).wait()
        pltpu.make_async_copy(
            lin_v.at[0, pl.ds(0, TAIL * FEATURES)],
            out_hbm.at[pl.ds(FULL_TILES * TILE * FEATURES, TAIL * FEATURES)],
            out_sem).start()
        pltpu.make_async_copy(
            lin_v.at[0, pl.ds(0, TAIL * FEATURES)],
            out_hbm.at[pl.ds(0, TAIL * FEATURES)], out_sem).wait()


def _lookup_body(idx_hbm, table_hbm, out_hbm, idx_v, rows_v, idx_sem,
                 gat_sem, out_sem):
    wid = lax.axis_index("s") * NUM_CORES + lax.axis_index("c")
    row0 = wid * ROWS_PER_W

    def start_idx(j, b):
        pltpu.make_async_copy(
            idx_hbm.at[pl.ds(row0 + j * CH_ROWS, CH_ROWS)],
            idx_v.at[b], idx_sem).start()

    def wait_idx(b):
        pltpu.make_async_copy(
            idx_hbm.at[pl.ds(row0, CH_ROWS)], idx_v.at[b], idx_sem).wait()

    def gather_fire(b):
        for r in range(CH_ROWS):
            pltpu.async_copy(table_hbm.at[idx_v.at[b, r]],
                             rows_v.at[b, r], gat_sem)

    def gather_wait(b):
        for r in range(CH_ROWS):
            pltpu.make_async_copy(table_hbm.at[idx_v.at[b, r]],
                                  rows_v.at[b, r], gat_sem).wait()

    def start_out(j, b):
        pltpu.make_async_copy(
            rows_v.at[b],
            out_hbm.at[pl.ds(row0 + j * CH_ROWS, CH_ROWS)],
            out_sem).start()

    def wait_out(b):
        pltpu.make_async_copy(
            rows_v.at[b], out_hbm.at[pl.ds(row0, CH_ROWS)], out_sem).wait()

    for b in range(NBUF):
        start_idx(b, b)
    wait_idx(0)
    gather_fire(0)

    def group(g, carry):
        for b in range(NBUF):
            j = NBUF * g + b
            nb = 1 - b

            @pl.when(j >= 1)
            def _(nb=nb):
                wait_out(nb)

            @pl.when(j + 1 < NCHUNK)
            def _(nb=nb):
                wait_idx(nb)
                gather_fire(nb)

            gather_wait(b)

            @pl.when(j + 2 < NCHUNK)
            def _(j=j, b=b):
                start_idx(j + 2, b)

            start_out(j, b)
        return carry

    lax.fori_loop(0, NCHUNK // NBUF, group, 0)

    wait_out((NCHUNK - 1) % NBUF)


@jax.jit
def _run(idx, embT, tail_flat):
    mesh = plsc.VectorSubcoreMesh(core_axis_name="c", subcore_axis_name="s")
    table_flat = pl.kernel(
        _compact_body,
        out_type=jax.ShapeDtypeStruct((NUM_EMB * FEATURES,), jnp.float32),
        mesh=mesh,
        scratch_types=[
            pltpu.VMEM((FEATURES, TILE), jnp.float32),
            pltpu.VMEM((2, TILE * FEATURES), jnp.float32),
            pltpu.SemaphoreType.DMA,
            pltpu.SemaphoreType.DMA,
        ],
        compiler_params=pltpu.CompilerParams(use_tc_tiling_on_sc=True,
                                             needs_layout_passes=False),
    )(embT, tail_flat)
    table32 = table_flat.reshape(NUM_EMB, FEATURES)
    out = pl.kernel(
        _lookup_body,
        out_type=jax.ShapeDtypeStruct((B_ROWS, SEQ, FEATURES), jnp.float32),
        mesh=mesh,
        scratch_types=[
            pltpu.VMEM((NBUF, CH_ROWS, SEQ), jnp.int32),
            pltpu.VMEM((NBUF, CH_ROWS, SEQ, FEATURES), jnp.float32),
            pltpu.SemaphoreType.DMA,
            pltpu.SemaphoreType.DMA,
            pltpu.SemaphoreType.DMA,
        ],
        compiler_params=pltpu.CompilerParams(use_tc_tiling_on_sc=False),
    )(idx, table32)
    return out


def kernel(inputs, embedding):
    tail_flat = embedding[FULL_TILES * TILE:].reshape(TAIL * FEATURES)
    return _run(inputs.astype(jnp.int32), embedding.T, tail_flat)


# speedup vs baseline: 1.0036x; 1.0036x over previous
import jax
import jax.numpy as jnp
from jax import lax
from jax.experimental import pallas as pl
from jax.experimental.pallas import tpu as pltpu
from jax.experimental.pallas import tpu_sc as plsc

NUM_CORES = 2
NUM_SUBCORES = 16
NW = NUM_CORES * NUM_SUBCORES

B_ROWS = 16384
SEQ = 50
FEATURES = 32
NUM_EMB = 1000000
LANES = 16

TILE = 128
FULL_TILES = NUM_EMB // TILE
TAIL = NUM_EMB - FULL_TILES * TILE
TILES_PER_W = -(-FULL_TILES // NW)

ROWS_PER_W = B_ROWS // NW
CH_ROWS = 16
CHUNK = CH_ROWS * SEQ
NCHUNK = ROWS_PER_W // CH_ROWS
NBUF = 2


def _compact_body(embT_hbm, tail_hbm, out_hbm, src_v, lin_v, in_sem, out_sem):
    wid = lax.axis_index("s") * NUM_CORES + lax.axis_index("c")
    t0 = wid * TILES_PER_W
    t1 = jnp.minimum(t0 + TILES_PER_W, FULL_TILES)
    iota = lax.iota(jnp.int32, LANES)

    def transpose_cols(nrows, lb):
        UNROLL = 8

        def col8(i8, carry):
            base = i8 * UNROLL
            for u in range(UNROLL):
                ic = base + u
                icv = jnp.full((LANES,), 0, jnp.int32) + ic
                for h in range(2):
                    vals = plsc.load_gather(src_v, [iota + h * LANES, icv])
                    lin_v[lb, pl.ds(ic * FEATURES + h * LANES, LANES)] = vals
            return carry
        lax.fori_loop(0, nrows // UNROLL, col8, 0)

    def body(i, carry):
        t = t0 + i

        @pl.when(t < t1)
        def _():
            lb = i % 2
            pltpu.make_async_copy(
                embT_hbm.at[:, pl.ds(t * TILE, TILE)], src_v,
                in_sem).start()
            pltpu.make_async_copy(
                embT_hbm.at[:, pl.ds(0, TILE)], src_v, in_sem).wait()
            transpose_cols(TILE, lb)

            @pl.when(i >= 2)
            def _():
                pltpu.make_async_copy(
                    lin_v.at[lb], out_hbm.at[pl.ds(0, TILE * FEATURES)],
                    out_sem).wait()

            pltpu.make_async_copy(
                lin_v.at[lb],
                out_hbm.at[pl.ds(t * TILE * FEATURES, TILE * FEATURES)],
                out_sem).start()
        return carry

    lax.fori_loop(0, TILES_PER_W, body, 0)

    n_tiles = t1 - t0

    @pl.when(n_tiles >= 2)
    def _():
        pltpu.make_async_copy(
            lin_v.at[0], out_hbm.at[pl.ds(0, TILE * FEATURES)],
            out_sem).wait()

    @pl.when(n_tiles >= 1)
    def _():
        pltpu.make_async_copy(
            lin_v.at[0], out_hbm.at[pl.ds(0, TILE * FEATURES)],
            out_sem).wait()

    @pl.when(wid == 0)
    def _():
        pltpu.make_async_copy(
            tail_hbm, lin_v.at[0, pl.ds(0, TAIL * FEATURES)],
            in_sem).start()
        pltpu.make_async_copy(
            tail_hbm, lin_v.at[0, pl.ds(0, TAIL * FEATURES)],
            in_sem).wait()
        pltpu.make_async_copy(
            lin_v.at[0, pl.ds(0, TAIL * FEATURES)],
            out_hbm.at[pl.ds(FULL_TILES * TILE * FEATURES, TAIL * FEATURES)],
            out_sem).start()
        pltpu.make_async_copy(
            lin_v.at[0, pl.ds(0, TAIL * FEATURES)],
            out_hbm.at[pl.ds(0, TAIL * FEATURES)], out_sem).wait()


def _lookup_body(idx_hbm, table_hbm, out_hbm, idx_v, rows_v, idx_sem,
                 gat_sem, out_sem):
    wid = lax.axis_index("s") * NUM_CORES + lax.axis_index("c")
    row0 = wid * ROWS_PER_W

    def start_idx(j, b):
        pltpu.make_async_copy(
            idx_hbm.at[pl.ds(row0 + j * CH_ROWS, CH_ROWS)],
            idx_v.at[b], idx_sem).start()

    def wait_idx(b):
        pltpu.make_async_copy(
            idx_hbm.at[pl.ds(row0, CH_ROWS)], idx_v.at[b], idx_sem).wait()

    def gather_fire(b):
        for r in range(CH_ROWS):
            pltpu.async_copy(table_hbm.at[idx_v.at[b, r]],
                             rows_v.at[b, r], gat_sem)

    def gather_wait(b):
        for r in range(CH_ROWS):
            pltpu.make_async_copy(table_hbm.at[idx_v.at[b, r]],
                                  rows_v.at[b, r], gat_sem).wait()

    def start_out(j, b):
        pltpu.make_async_copy(
            rows_v.at[b],
            out_hbm.at[pl.ds(row0 + j * CH_ROWS, CH_ROWS)],
            out_sem).start()

    def wait_out(b):
        pltpu.make_async_copy(
            rows_v.at[b], out_hbm.at[pl.ds(row0, CH_ROWS)], out_sem).wait()

    for b in range(NBUF):
        start_idx(b, b)
    wait_idx(0)
    gather_fire(0)

    def group(g, carry):
        for b in range(NBUF):
            j = NBUF * g + b
            nb = 1 - b

            @pl.when(j >= 1)
            def _(nb=nb):
                wait_out(nb)

            @pl.when(j + 1 < NCHUNK)
            def _(nb=nb):
                wait_idx(nb)
                gather_fire(nb)

            gather_wait(b)

            @pl.when(j + 2 < NCHUNK)
            def _(j=j, b=b):
                start_idx(j + 2, b)

            start_out(j, b)
        return carry

    lax.fori_loop(0, NCHUNK // NBUF, group, 0)

    wait_out((NCHUNK - 1) % NBUF)


@jax.jit
def _run(idx, embT, tail_flat):
    mesh = plsc.VectorSubcoreMesh(core_axis_name="c", subcore_axis_name="s")
    table_flat = pl.kernel(
        _compact_body,
        out_type=jax.ShapeDtypeStruct((NUM_EMB * FEATURES,), jnp.float32),
        mesh=mesh,
        scratch_types=[
            pltpu.VMEM((FEATURES, TILE), jnp.float32),
            pltpu.VMEM((2, TILE * FEATURES), jnp.float32),
            pltpu.SemaphoreType.DMA,
            pltpu.SemaphoreType.DMA,
        ],
        compiler_params=pltpu.CompilerParams(use_tc_tiling_on_sc=True,
                                             needs_layout_passes=False),
    )(embT, tail_flat)
    table32 = table_flat.reshape(NUM_EMB, FEATURES)
    out = pl.kernel(
        _lookup_body,
        out_type=jax.ShapeDtypeStruct((B_ROWS, SEQ, FEATURES), jnp.float32),
        mesh=mesh,
        scratch_types=[
            pltpu.VMEM((NBUF, CH_ROWS, SEQ), jnp.int32),
            pltpu.VMEM((NBUF, CH_ROWS, SEQ, FEATURES), jnp.float32),
            pltpu.SemaphoreType.DMA,
            pltpu.SemaphoreType.DMA,
            pltpu.SemaphoreType.DMA,
        ],
        compiler_params=pltpu.CompilerParams(use_tc_tiling_on_sc=False),
    )(idx, table32)
    return out


def kernel(inputs, embedding):
    tail_flat = embedding[FULL_TILES * TILE:].reshape(TAIL * FEATURES)
    return _run(inputs.astype(jnp.int32), embedding.T, tail_flat)


# speedup vs baseline: 1.1507x; 1.1466x over previous
import jax
import jax.numpy as jnp
from jax import lax
from jax.experimental import pallas as pl
from jax.experimental.pallas import tpu as pltpu
from jax.experimental.pallas import tpu_sc as plsc

NUM_CORES = 2
NUM_SUBCORES = 16
NW = NUM_CORES * NUM_SUBCORES

B_ROWS = 16384
SEQ = 50
FEATURES = 32
NUM_EMB = 1000000
LANES = 16

TILE = 128
FULL_TILES = NUM_EMB // TILE
TAIL = NUM_EMB - FULL_TILES * TILE
TILES_PER_W = -(-FULL_TILES // NW)

ROWS_PER_W = B_ROWS // NW
CH_ROWS = 16
CHUNK = CH_ROWS * SEQ
NCHUNK = ROWS_PER_W // CH_ROWS
NBUF = 2


def _compact_body(embT_hbm, tail_hbm, out_hbm, src_v, lin_v, in_sem, out_sem):
    wid = lax.axis_index("s") * NUM_CORES + lax.axis_index("c")
    t0 = wid * TILES_PER_W
    t1 = jnp.minimum(t0 + TILES_PER_W, FULL_TILES)
    iota = lax.iota(jnp.int32, LANES)

    def transpose_cols(nrows, lb):
        UNROLL = 8

        def col8(i8, carry):
            base = i8 * UNROLL
            for u in range(UNROLL):
                ic = base + u
                icv = jnp.full((LANES,), 0, jnp.int32) + ic
                for h in range(2):
                    vals = plsc.load_gather(src_v.at[lb],
                                            [iota + h * LANES, icv])
                    lin_v[lb, pl.ds(ic * FEATURES + h * LANES, LANES)] = vals
            return carry
        lax.fori_loop(0, nrows // UNROLL, col8, 0)

    def start_in(t, sb):
        pltpu.make_async_copy(
            embT_hbm.at[:, pl.ds(t * TILE, TILE)], src_v.at[sb],
            in_sem).start()

    def wait_in(sb):
        pltpu.make_async_copy(
            embT_hbm.at[:, pl.ds(0, TILE)], src_v.at[sb], in_sem).wait()

    start_in(t0, 0)

    def body(i, carry):
        t = t0 + i

        @pl.when(t < t1)
        def _(i=i, t=t):
            lb = i % 2

            @pl.when(t + 1 < t1)
            def _():
                start_in(t + 1, 1 - lb)

            wait_in(lb)
            transpose_cols(TILE, lb)

            @pl.when(i >= 2)
            def _():
                pltpu.make_async_copy(
                    lin_v.at[lb], out_hbm.at[pl.ds(0, TILE * FEATURES)],
                    out_sem).wait()

            pltpu.make_async_copy(
                lin_v.at[lb],
                out_hbm.at[pl.ds(t * TILE * FEATURES, TILE * FEATURES)],
                out_sem).start()
        return carry

    lax.fori_loop(0, TILES_PER_W, body, 0)

    n_tiles = t1 - t0

    @pl.when(n_tiles >= 2)
    def _():
        pltpu.make_async_copy(
            lin_v.at[0], out_hbm.at[pl.ds(0, TILE * FEATURES)],
            out_sem).wait()

    @pl.when(n_tiles >= 1)
    def _():
        pltpu.make_async_copy(
            lin_v.at[0], out_hbm.at[pl.ds(0, TILE * FEATURES)],
            out_sem).wait()

    @pl.when(wid == 0)
    def _():
        pltpu.make_async_copy(
            tail_hbm, lin_v.at[0, pl.ds(0, TAIL * FEATURES)],
            in_sem).start()
        pltpu.make_async_copy(
            tail_hbm, lin_v.at[0, pl.ds(0, TAIL * FEATURES)],
            in_sem).wait()
        pltpu.make_async_copy(
            lin_v.at[0, pl.ds(0, TAIL * FEATURES)],
            out_hbm.at[pl.ds(FULL_TILES * TILE * FEATURES, TAIL * FEATURES)],
            out_sem).start()
        pltpu.make_async_copy(
            lin_v.at[0, pl.ds(0, TAIL * FEATURES)],
            out_hbm.at[pl.ds(0, TAIL * FEATURES)], out_sem).wait()


def _lookup_body(idx_hbm, table_hbm, out_hbm, idx_v, rows_v, idx_sem,
                 gat_sem, out_sem):
    wid = lax.axis_index("s") * NUM_CORES + lax.axis_index("c")
    row0 = wid * ROWS_PER_W

    def start_idx(j, b):
        pltpu.make_async_copy(
            idx_hbm.at[pl.ds(row0 + j * CH_ROWS, CH_ROWS)],
            idx_v.at[b], idx_sem).start()

    def wait_idx(b):
        pltpu.make_async_copy(
            idx_hbm.at[pl.ds(row0, CH_ROWS)], idx_v.at[b], idx_sem).wait()

    def gather_fire(b):
        for r in range(CH_ROWS):
            pltpu.async_copy(table_hbm.at[idx_v.at[b, r]],
                             rows_v.at[b, r], gat_sem)

    def gather_wait(b):
        for r in range(CH_ROWS):
            pltpu.make_async_copy(table_hbm.at[idx_v.at[b, r]],
                                  rows_v.at[b, r], gat_sem).wait()

    def start_out(j, b):
        pltpu.make_async_copy(
            rows_v.at[b],
            out_hbm.at[pl.ds(row0 + j * CH_ROWS, CH_ROWS)],
            out_sem).start()

    def wait_out(b):
        pltpu.make_async_copy(
            rows_v.at[b], out_hbm.at[pl.ds(row0, CH_ROWS)], out_sem).wait()

    for b in range(NBUF):
        start_idx(b, b)
    wait_idx(0)
    gather_fire(0)

    def group(g, carry):
        for b in range(NBUF):
            j = NBUF * g + b
            nb = 1 - b

            @pl.when(j >= 1)
            def _(nb=nb):
                wait_out(nb)

            @pl.when(j + 1 < NCHUNK)
            def _(nb=nb):
                wait_idx(nb)
                gather_fire(nb)

            gather_wait(b)

            @pl.when(j + 2 < NCHUNK)
            def _(j=j, b=b):
                start_idx(j + 2, b)

            start_out(j, b)
        return carry

    lax.fori_loop(0, NCHUNK // NBUF, group, 0)

    wait_out((NCHUNK - 1) % NBUF)


@jax.jit
def _run(idx, embT, tail_flat):
    mesh = plsc.VectorSubcoreMesh(core_axis_name="c", subcore_axis_name="s")
    table_flat = pl.kernel(
        _compact_body,
        out_type=jax.ShapeDtypeStruct((NUM_EMB * FEATURES,), jnp.float32),
        mesh=mesh,
        scratch_types=[
            pltpu.VMEM((2, FEATURES, TILE), jnp.float32),
            pltpu.VMEM((2, TILE * FEATURES), jnp.float32),
            pltpu.SemaphoreType.DMA,
            pltpu.SemaphoreType.DMA,
        ],
        compiler_params=pltpu.CompilerParams(use_tc_tiling_on_sc=True,
                                             needs_layout_passes=False),
    )(embT, tail_flat)
    table32 = table_flat.reshape(NUM_EMB, FEATURES)
    out = pl.kernel(
        _lookup_body,
        out_type=jax.ShapeDtypeStruct((B_ROWS, SEQ, FEATURES), jnp.float32),
        mesh=mesh,
        scratch_types=[
            pltpu.VMEM((NBUF, CH_ROWS, SEQ), jnp.int32),
            pltpu.VMEM((NBUF, CH_ROWS, SEQ, FEATURES), jnp.float32),
            pltpu.SemaphoreType.DMA,
            pltpu.SemaphoreType.DMA,
            pltpu.SemaphoreType.DMA,
        ],
        compiler_params=pltpu.CompilerParams(use_tc_tiling_on_sc=False),
    )(idx, table32)
    return out


def kernel(inputs, embedding):
    tail_flat = embedding[FULL_TILES * TILE:].reshape(TAIL * FEATURES)
    return _run(inputs.astype(jnp.int32), embedding.T, tail_flat)


# speedup vs baseline: 1.4933x; 1.2977x over previous
import jax
import jax.numpy as jnp
from jax import lax
from jax.experimental import pallas as pl
from jax.experimental.pallas import tpu as pltpu
from jax.experimental.pallas import tpu_sc as plsc

NUM_CORES = 2
NUM_SUBCORES = 16
NW = NUM_CORES * NUM_SUBCORES

B_ROWS = 16384
SEQ = 50
FEATURES = 32
ROWS_PER_W = B_ROWS // NW
CH_ROWS = 16
NCHUNK = ROWS_PER_W // CH_ROWS
NBUF = 2


def _body(idx_hbm, table_hbm, out_hbm, idx_v, rows_v, idx_sem, gat_sem,
          out_sem):
    wid = lax.axis_index("s") * NUM_CORES + lax.axis_index("c")
    row0 = wid * ROWS_PER_W

    def start_idx(j, b):
        pltpu.make_async_copy(
            idx_hbm.at[pl.ds(row0 + j * CH_ROWS, CH_ROWS)],
            idx_v.at[b], idx_sem).start()

    def wait_idx(b):
        pltpu.make_async_copy(
            idx_hbm.at[pl.ds(row0, CH_ROWS)], idx_v.at[b], idx_sem).wait()

    def gather_fire(b):
        for r in range(CH_ROWS):
            pltpu.async_copy(table_hbm.at[idx_v.at[b, r]],
                             rows_v.at[b, r], gat_sem)

    def gather_wait(b):
        for r in range(CH_ROWS):
            pltpu.make_async_copy(table_hbm.at[idx_v.at[b, r]],
                                  rows_v.at[b, r], gat_sem).wait()

    def start_out(j, b):
        pltpu.make_async_copy(
            rows_v.at[b],
            out_hbm.at[pl.ds(row0 + j * CH_ROWS, CH_ROWS)],
            out_sem).start()

    def wait_out(b):
        pltpu.make_async_copy(
            rows_v.at[b], out_hbm.at[pl.ds(row0, CH_ROWS)], out_sem).wait()

    for b in range(NBUF):
        start_idx(b, b)
    wait_idx(0)
    gather_fire(0)

    def group(g, carry):
        for b in range(NBUF):
            j = NBUF * g + b
            nb = 1 - b

            @pl.when(j >= 1)
            def _(nb=nb):
                wait_out(nb)

            @pl.when(j + 1 < NCHUNK)
            def _(nb=nb):
                wait_idx(nb)
                gather_fire(nb)

            gather_wait(b)

            @pl.when(j + 2 < NCHUNK)
            def _(j=j, b=b):
                start_idx(j + 2, b)

            start_out(j, b)
        return carry

    lax.fori_loop(0, NCHUNK // NBUF, group, 0)

    wait_out((NCHUNK - 1) % NBUF)


@jax.jit
def _lookup(idx, embedding):
    mesh = plsc.VectorSubcoreMesh(core_axis_name="c", subcore_axis_name="s")
    run = pl.kernel(
        _body,
        out_type=jax.ShapeDtypeStruct((B_ROWS, SEQ, FEATURES), jnp.float32),
        mesh=mesh,
        scratch_types=[
            pltpu.VMEM((NBUF, CH_ROWS, SEQ), jnp.int32),
            pltpu.VMEM((NBUF, CH_ROWS, SEQ, FEATURES), jnp.float32),
            pltpu.SemaphoreType.DMA,
            pltpu.SemaphoreType.DMA,
            pltpu.SemaphoreType.DMA,
        ],
        compiler_params=pltpu.CompilerParams(use_tc_tiling_on_sc=False),
    )
    return run(idx, embedding)


def kernel(inputs, embedding):
    return _lookup(inputs.astype(jnp.int32), embedding)
